# baseline (speedup 1.0000x reference)
import functools

import jax
import jax.numpy as jnp
from jax import lax
from jax.experimental import pallas as pl
from jax.experimental.pallas import tpu as pltpu
from jax.experimental.pallas import tpu_sc as plsc

_B = 16384
_D = 64
_NC = 2
_NS = 16
_NW = _NC * _NS
_CHUNK = 128
_ROWS_PER_W = _B // _NW
_CHUNKS_PER_W = _ROWS_PER_W // _CHUNK


def _sc_gather(user2d, item2d, user_table, item_table):
    mesh = plsc.VectorSubcoreMesh(core_axis_name="c", subcore_axis_name="s")

    @functools.partial(
        pl.kernel,
        mesh=mesh,
        compiler_params=pltpu.CompilerParams(use_tc_tiling_on_sc=False),
        out_type=[
            jax.ShapeDtypeStruct((_B, _D), jnp.float32),
            jax.ShapeDtypeStruct((_B, _D), jnp.float32),
        ],
        scratch_types=[
            pltpu.VMEM((_CHUNKS_PER_W, _CHUNK), jnp.int32),
            pltpu.VMEM((_CHUNKS_PER_W, _CHUNK), jnp.int32),
            pltpu.VMEM((_ROWS_PER_W, _D), jnp.float32),
            pltpu.VMEM((_ROWS_PER_W, _D), jnp.float32),
            pltpu.SemaphoreType.DMA,
            pltpu.SemaphoreType.DMA,
        ],
    )
    def gather_kernel(uidx_hbm, iidx_hbm, utab_hbm, itab_hbm, u_out, i_out,
                      uidx_v, iidx_v, urows_v, irows_v, usem, isem):
        wid = lax.axis_index("s") * _NC + lax.axis_index("c")
        row0 = wid * _CHUNKS_PER_W
        pltpu.sync_copy(uidx_hbm.at[pl.ds(row0, _CHUNKS_PER_W)], uidx_v)
        pltpu.sync_copy(iidx_hbm.at[pl.ds(row0, _CHUNKS_PER_W)], iidx_v)
        copies = []
        for k in range(_CHUNKS_PER_W):
            dst = pl.ds(k * _CHUNK, _CHUNK)
            copies.append(
                pltpu.async_copy(utab_hbm.at[uidx_v.at[k]], urows_v.at[dst], usem))
            copies.append(
                pltpu.async_copy(itab_hbm.at[iidx_v.at[k]], irows_v.at[dst], isem))
        for c in copies:
            c.wait()
        base = wid * _ROWS_PER_W
        pltpu.sync_copy(urows_v, u_out.at[pl.ds(base, _ROWS_PER_W)])
        pltpu.sync_copy(irows_v, i_out.at[pl.ds(base, _ROWS_PER_W)])

    return gather_kernel(user2d, item2d, user_table, item_table)


_T = 2048


def _mlp_body(u_ref, i_ref, w1u_ref, w1i_ref, b1_ref, w2_ref, b2_ref,
              w3_ref, b3_ref, o_ref):
    h = jnp.dot(u_ref[...], w1u_ref[...], preferred_element_type=jnp.float32)
    h = h + jnp.dot(i_ref[...], w1i_ref[...], preferred_element_type=jnp.float32)
    h = jnp.maximum(h + b1_ref[...], 0.0)
    h2 = jnp.dot(h, w2_ref[...], preferred_element_type=jnp.float32)
    h2 = jnp.maximum(h2 + b2_ref[...], 0.0)
    o_ref[...] = jnp.sum(h2 * w3_ref[...], axis=1) + b3_ref[0, 0]


def _tc_mlp(u, i, W1, b1, W2, b2, W3, b3):
    w1u = W1[:, :_D].T
    w1i = W1[:, _D:].T
    b1r = b1.reshape(1, 128)
    w2t = W2.T
    b2r = b2.reshape(1, 64)
    w3r = W3.reshape(1, 64)
    b3r = b3.reshape(1, 1)
    grid = (_B // _T,)
    full = lambda shape: pl.BlockSpec(shape, lambda b: (0, 0))
    return pl.pallas_call(
        _mlp_body,
        grid=grid,
        in_specs=[
            pl.BlockSpec((_T, _D), lambda b: (b, 0)),
            pl.BlockSpec((_T, _D), lambda b: (b, 0)),
            full((_D, 128)),
            full((_D, 128)),
            full((1, 128)),
            full((128, _D)),
            full((1, _D)),
            full((1, _D)),
            full((1, 1)),
        ],
        out_specs=pl.BlockSpec((_T,), lambda b: (b,)),
        out_shape=jax.ShapeDtypeStruct((_B,), jnp.float32),
    )(u, i, w1u, w1i, b1r, w2t, b2r, w3r, b3r)


def kernel(user, item, user_table, item_table, W1, b1, W2, b2, W3, b3):
    user2d = user.reshape(_B // _CHUNK, _CHUNK)
    item2d = item.reshape(_B // _CHUNK, _CHUNK)
    u, i = _sc_gather(user2d, item2d, user_table, item_table)
    return _tc_mlp(u, i, W1, b1, W2, b2, W3, b3)


# speedup vs baseline: 1.6325x; 1.6325x over previous
import functools

import jax
import jax.numpy as jnp
from jax import lax
from jax.experimental import pallas as pl
from jax.experimental.pallas import tpu as pltpu
from jax.experimental.pallas import tpu_sc as plsc

_B = 16384
_D = 64
_NC = 2
_NS = 16
_NW = _NC * _NS
_ROWS_PER_W = _B // _NW


def _sc_gather(user, item, user_table, item_table):
    mesh = plsc.VectorSubcoreMesh(core_axis_name="c", subcore_axis_name="s")

    @functools.partial(
        pl.kernel,
        mesh=mesh,
        compiler_params=pltpu.CompilerParams(use_tc_tiling_on_sc=True),
        out_type=[
            jax.ShapeDtypeStruct((_B, _D), jnp.float32),
            jax.ShapeDtypeStruct((_B, _D), jnp.float32),
        ],
        scratch_types=[
            pltpu.VMEM((_ROWS_PER_W,), jnp.int32),
            pltpu.VMEM((_ROWS_PER_W,), jnp.int32),
            pltpu.VMEM((_ROWS_PER_W // 2, _D), jnp.float32),
            pltpu.VMEM((_ROWS_PER_W // 2, _D), jnp.float32),
            pltpu.SemaphoreType.DMA,
            pltpu.SemaphoreType.DMA,
        ],
    )
    def gather_kernel(uidx_hbm, iidx_hbm, utab_hbm, itab_hbm, u_out, i_out,
                      uidx_v, iidx_v, urows_v, irows_v, usem, isem):
        wid = lax.axis_index("s") * _NC + lax.axis_index("c")
        base = wid * _ROWS_PER_W
        half = _ROWS_PER_W // 2
        pltpu.sync_copy(uidx_hbm.at[pl.ds(base, _ROWS_PER_W)], uidx_v)
        pltpu.sync_copy(iidx_hbm.at[pl.ds(base, _ROWS_PER_W)], iidx_v)

        for p in range(2):
            def issue(g, _):
                uvec = uidx_v[pl.ds(p * half + g * 16, 16)]
                ivec = iidx_v[pl.ds(p * half + g * 16, 16)]
                for j in range(16):
                    pltpu.async_copy(utab_hbm.at[pl.ds(uvec[j], 1)],
                                     urows_v.at[pl.ds(g * 16 + j, 1)], usem)
                    pltpu.async_copy(itab_hbm.at[pl.ds(ivec[j], 1)],
                                     irows_v.at[pl.ds(g * 16 + j, 1)], isem)
                return 0

            lax.fori_loop(0, half // 16, issue, 0)

            def drain(i, _):
                pltpu.make_async_copy(utab_hbm.at[pl.ds(0, 1)],
                                      urows_v.at[pl.ds(i, 1)], usem).wait()
                pltpu.make_async_copy(itab_hbm.at[pl.ds(0, 1)],
                                      irows_v.at[pl.ds(i, 1)], isem).wait()
                return 0

            lax.fori_loop(0, half, drain, 0)
            pltpu.sync_copy(urows_v, u_out.at[pl.ds(base + p * half, half)])
            pltpu.sync_copy(irows_v, i_out.at[pl.ds(base + p * half, half)])

    return gather_kernel(user, item, user_table, item_table)


_T = 2048


def _mlp_body(u_ref, i_ref, w1u_ref, w1i_ref, b1_ref, w2_ref, b2_ref,
              w3_ref, b3_ref, o_ref):
    h = jnp.dot(u_ref[...], w1u_ref[...], preferred_element_type=jnp.float32)
    h = h + jnp.dot(i_ref[...], w1i_ref[...], preferred_element_type=jnp.float32)
    h = jnp.maximum(h + b1_ref[...], 0.0)
    h2 = jnp.dot(h, w2_ref[...], preferred_element_type=jnp.float32)
    h2 = jnp.maximum(h2 + b2_ref[...], 0.0)
    o_ref[...] = jnp.sum(h2 * w3_ref[...], axis=1) + b3_ref[0, 0]


def _tc_mlp(u, i, W1, b1, W2, b2, W3, b3):
    w1u = W1[:, :_D].T
    w1i = W1[:, _D:].T
    b1r = b1.reshape(1, 128)
    w2t = W2.T
    b2r = b2.reshape(1, 64)
    w3r = W3.reshape(1, 64)
    b3r = b3.reshape(1, 1)
    grid = (_B // _T,)
    full = lambda shape: pl.BlockSpec(shape, lambda b: (0, 0))
    return pl.pallas_call(
        _mlp_body,
        grid=grid,
        in_specs=[
            pl.BlockSpec((_T, _D), lambda b: (b, 0)),
            pl.BlockSpec((_T, _D), lambda b: (b, 0)),
            full((_D, 128)),
            full((_D, 128)),
            full((1, 128)),
            full((128, _D)),
            full((1, _D)),
            full((1, _D)),
            full((1, 1)),
        ],
        out_specs=pl.BlockSpec((_T,), lambda b: (b,)),
        out_shape=jax.ShapeDtypeStruct((_B,), jnp.float32),
    )(u, i, w1u, w1i, b1r, w2t, b2r, w3r, b3r)


def kernel(user, item, user_table, item_table, W1, b1, W2, b2, W3, b3):
    u, i = _sc_gather(user, item, user_table, item_table)
    return _tc_mlp(u, i, W1, b1, W2, b2, W3, b3)


# speedup vs baseline: 2.3559x; 1.4432x over previous
import functools

import jax
import jax.numpy as jnp
from jax import lax
from jax.experimental import pallas as pl
from jax.experimental.pallas import tpu as pltpu
from jax.experimental.pallas import tpu_sc as plsc

_B = 16384
_D = 64
_NC = 2
_NS = 16
_NW = _NC * _NS
_ROWS_PER_W = _B // _NW


def _sc_gather(user, item, utab3, itab3):
    mesh = plsc.VectorSubcoreMesh(core_axis_name="c", subcore_axis_name="s")
    half = _ROWS_PER_W // 2

    @functools.partial(
        pl.kernel,
        mesh=mesh,
        compiler_params=pltpu.CompilerParams(use_tc_tiling_on_sc=True),
        out_type=[
            jax.ShapeDtypeStruct((_B, _D), jnp.float32),
            jax.ShapeDtypeStruct((_B, _D), jnp.float32),
        ],
        scratch_types=[
            pltpu.VMEM((_ROWS_PER_W,), jnp.int32),
            pltpu.VMEM((_ROWS_PER_W,), jnp.int32),
            pltpu.VMEM((half, _D), jnp.float32),
            pltpu.VMEM((half, _D), jnp.float32),
            pltpu.SemaphoreType.DMA,
            pltpu.SemaphoreType.DMA,
        ],
    )
    def gather_kernel(uidx_hbm, iidx_hbm, utab_hbm, itab_hbm, u_out, i_out,
                      uidx_v, iidx_v, urows_v, irows_v, usem, isem):
        wid = lax.axis_index("s") * _NC + lax.axis_index("c")
        base = wid * _ROWS_PER_W
        pltpu.sync_copy(uidx_hbm.at[pl.ds(base, _ROWS_PER_W)], uidx_v)
        pltpu.sync_copy(iidx_hbm.at[pl.ds(base, _ROWS_PER_W)], iidx_v)

        for p in range(2):
            def issue(g, _):
                uvec = uidx_v[pl.ds(p * half + g * 16, 16)]
                ivec = iidx_v[pl.ds(p * half + g * 16, 16)]
                for j in range(16):
                    r = uvec[j]
                    pltpu.async_copy(
                        utab_hbm.at[r >> 3, pl.ds(r & 7, 1)],
                        urows_v.at[pl.ds(g * 16 + j, 1)], usem)
                    r2 = ivec[j]
                    pltpu.async_copy(
                        itab_hbm.at[r2 >> 3, pl.ds(r2 & 7, 1)],
                        irows_v.at[pl.ds(g * 16 + j, 1)], isem)
                return 0

            lax.fori_loop(0, half // 16, issue, 0)

            def drain(i, _):
                pltpu.make_async_copy(utab_hbm.at[0, pl.ds(0, 1)],
                                      urows_v.at[pl.ds(i, 1)], usem).wait()
                pltpu.make_async_copy(itab_hbm.at[0, pl.ds(0, 1)],
                                      irows_v.at[pl.ds(i, 1)], isem).wait()
                return 0

            lax.fori_loop(0, half, drain, 0)
            pltpu.sync_copy(urows_v, u_out.at[pl.ds(base + p * half, half)])
            pltpu.sync_copy(irows_v, i_out.at[pl.ds(base + p * half, half)])

    return gather_kernel(user, item, utab3, itab3)


_T = 2048


def _mlp_body(u_ref, i_ref, w1u_ref, w1i_ref, b1_ref, w2_ref, b2_ref,
              w3_ref, b3_ref, o_ref):
    h = jnp.dot(u_ref[...], w1u_ref[...], preferred_element_type=jnp.float32)
    h = h + jnp.dot(i_ref[...], w1i_ref[...], preferred_element_type=jnp.float32)
    h = jnp.maximum(h + b1_ref[...], 0.0)
    h2 = jnp.dot(h, w2_ref[...], preferred_element_type=jnp.float32)
    h2 = jnp.maximum(h2 + b2_ref[...], 0.0)
    o_ref[...] = jnp.sum(h2 * w3_ref[...], axis=1) + b3_ref[0, 0]


def _tc_mlp(u, i, W1, b1, W2, b2, W3, b3):
    w1u = W1[:, :_D].T
    w1i = W1[:, _D:].T
    b1r = b1.reshape(1, 128)
    w2t = W2.T
    b2r = b2.reshape(1, 64)
    w3r = W3.reshape(1, 64)
    b3r = b3.reshape(1, 1)
    grid = (_B // _T,)
    full = lambda shape: pl.BlockSpec(shape, lambda b: (0, 0))
    return pl.pallas_call(
        _mlp_body,
        grid=grid,
        in_specs=[
            pl.BlockSpec((_T, _D), lambda b: (b, 0)),
            pl.BlockSpec((_T, _D), lambda b: (b, 0)),
            full((_D, 128)),
            full((_D, 128)),
            full((1, 128)),
            full((128, _D)),
            full((1, _D)),
            full((1, _D)),
            full((1, 1)),
        ],
        out_specs=pl.BlockSpec((_T,), lambda b: (b,)),
        out_shape=jax.ShapeDtypeStruct((_B,), jnp.float32),
    )(u, i, w1u, w1i, b1r, w2t, b2r, w3r, b3r)


def kernel(user, item, user_table, item_table, W1, b1, W2, b2, W3, b3):
    utab3 = user_table.reshape(125000, 8, _D)
    itab3 = item_table.reshape(12500, 8, _D)
    u, i = _sc_gather(user, item, utab3, itab3)
    return _tc_mlp(u, i, W1, b1, W2, b2, W3, b3)


# speedup vs baseline: 2.4230x; 1.0285x over previous
import functools

import jax
import jax.numpy as jnp
from jax import lax
from jax.experimental import pallas as pl
from jax.experimental.pallas import tpu as pltpu
from jax.experimental.pallas import tpu_sc as plsc

_B = 16384
_D = 64
_NC = 2
_NS = 16
_NW = _NC * _NS
_ROWS_PER_W = _B // _NW


def _sc_gather(user, item, utab3, itab3):
    mesh = plsc.VectorSubcoreMesh(core_axis_name="c", subcore_axis_name="s")
    half = _ROWS_PER_W // 2

    @functools.partial(
        pl.kernel,
        mesh=mesh,
        compiler_params=pltpu.CompilerParams(use_tc_tiling_on_sc=True),
        out_type=[
            jax.ShapeDtypeStruct((_B, _D), jnp.float32),
            jax.ShapeDtypeStruct((_B, _D), jnp.float32),
        ],
        scratch_types=[
            pltpu.VMEM((_ROWS_PER_W,), jnp.int32),
            pltpu.VMEM((_ROWS_PER_W,), jnp.int32),
            pltpu.VMEM((half, _D), jnp.float32),
            pltpu.VMEM((half, _D), jnp.float32),
            pltpu.SemaphoreType.DMA,
            pltpu.SemaphoreType.DMA,
        ],
    )
    def gather_kernel(uidx_hbm, iidx_hbm, utab_hbm, itab_hbm, u_out, i_out,
                      uidx_v, iidx_v, urows_v, irows_v, usem, isem):
        wid = lax.axis_index("s") * _NC + lax.axis_index("c")
        base = wid * _ROWS_PER_W
        pltpu.sync_copy(uidx_hbm.at[pl.ds(base, _ROWS_PER_W)], uidx_v)
        pltpu.sync_copy(iidx_hbm.at[pl.ds(base, _ROWS_PER_W)], iidx_v)

        for p in range(2):
            def issue(g, _):
                uvec = uidx_v[pl.ds(p * half + g * 16, 16)]
                ivec = iidx_v[pl.ds(p * half + g * 16, 16)]
                for j in range(16):
                    r = uvec[j]
                    pltpu.async_copy(
                        utab_hbm.at[r >> 3, pl.ds(r & 7, 1)],
                        urows_v.at[pl.ds(g * 16 + j, 1)], usem)
                    r2 = ivec[j]
                    pltpu.async_copy(
                        itab_hbm.at[pl.ds(r2, 1)],
                        irows_v.at[pl.ds(g * 16 + j, 1)], isem)
                return 0

            lax.fori_loop(0, half // 16, issue, 0)

            def drain(i, _):
                pltpu.make_async_copy(utab_hbm.at[0, pl.ds(0, 1)],
                                      urows_v.at[pl.ds(i, 1)], usem).wait()
                pltpu.make_async_copy(itab_hbm.at[pl.ds(0, 1)],
                                      irows_v.at[pl.ds(i, 1)], isem).wait()
                return 0

            lax.fori_loop(0, half, drain, 0)
            pltpu.sync_copy(urows_v, u_out.at[pl.ds(base + p * half, half)])
            pltpu.sync_copy(irows_v, i_out.at[pl.ds(base + p * half, half)])

    return gather_kernel(user, item, utab3, itab3)


_T = 2048


def _mlp_body(u_ref, i_ref, w1u_ref, w1i_ref, b1_ref, w2_ref, b2_ref,
              w3_ref, b3_ref, o_ref):
    h = jnp.dot(u_ref[...], w1u_ref[...], preferred_element_type=jnp.float32)
    h = h + jnp.dot(i_ref[...], w1i_ref[...], preferred_element_type=jnp.float32)
    h = jnp.maximum(h + b1_ref[...], 0.0)
    h2 = jnp.dot(h, w2_ref[...], preferred_element_type=jnp.float32)
    h2 = jnp.maximum(h2 + b2_ref[...], 0.0)
    o_ref[...] = jnp.sum(h2 * w3_ref[...], axis=1) + b3_ref[0, 0]


def _tc_mlp(u, i, W1, b1, W2, b2, W3, b3):
    w1u = W1[:, :_D].T
    w1i = W1[:, _D:].T
    b1r = b1.reshape(1, 128)
    w2t = W2.T
    b2r = b2.reshape(1, 64)
    w3r = W3.reshape(1, 64)
    b3r = b3.reshape(1, 1)
    grid = (_B // _T,)
    full = lambda shape: pl.BlockSpec(shape, lambda b: (0, 0))
    return pl.pallas_call(
        _mlp_body,
        grid=grid,
        in_specs=[
            pl.BlockSpec((_T, _D), lambda b: (b, 0)),
            pl.BlockSpec((_T, _D), lambda b: (b, 0)),
            full((_D, 128)),
            full((_D, 128)),
            full((1, 128)),
            full((128, _D)),
            full((1, _D)),
            full((1, _D)),
            full((1, 1)),
        ],
        out_specs=pl.BlockSpec((_T,), lambda b: (b,)),
        out_shape=jax.ShapeDtypeStruct((_B,), jnp.float32),
    )(u, i, w1u, w1i, b1r, w2t, b2r, w3r, b3r)


def kernel(user, item, user_table, item_table, W1, b1, W2, b2, W3, b3):
    utab3 = user_table.reshape(125000, 8, _D)
    u, i = _sc_gather(user, item, utab3, item_table)
    return _tc_mlp(u, i, W1, b1, W2, b2, W3, b3)


# speedup vs baseline: 2.4614x; 1.0159x over previous
import functools

import jax
import jax.numpy as jnp
from jax import lax
from jax.experimental import pallas as pl
from jax.experimental.pallas import tpu as pltpu
from jax.experimental.pallas import tpu_sc as plsc

_B = 16384
_D = 64
_NC = 2
_NS = 16
_NW = _NC * _NS
_ROWS_PER_W = _B // _NW


def _sc_gather(user, item, utab3, itab3):
    mesh = plsc.VectorSubcoreMesh(core_axis_name="c", subcore_axis_name="s")
    half = _ROWS_PER_W // 2

    @functools.partial(
        pl.kernel,
        mesh=mesh,
        compiler_params=pltpu.CompilerParams(use_tc_tiling_on_sc=True),
        out_type=[
            jax.ShapeDtypeStruct((_B, _D), jnp.float32),
            jax.ShapeDtypeStruct((_B, _D), jnp.float32),
        ],
        scratch_types=[
            pltpu.VMEM((_ROWS_PER_W,), jnp.int32),
            pltpu.VMEM((_ROWS_PER_W,), jnp.int32),
            pltpu.VMEM((half, _D), jnp.float32),
            pltpu.VMEM((half, _D), jnp.float32),
            pltpu.SemaphoreType.DMA,
            pltpu.SemaphoreType.DMA,
        ],
    )
    def gather_kernel(uidx_hbm, iidx_hbm, utab_hbm, itab_hbm, u_out, i_out,
                      uidx_v, iidx_v, urows_v, irows_v, usem, isem):
        wid = lax.axis_index("s") * _NC + lax.axis_index("c")
        base = wid * _ROWS_PER_W
        pltpu.sync_copy(uidx_hbm.at[pl.ds(base, _ROWS_PER_W)], uidx_v)
        pltpu.sync_copy(iidx_hbm.at[pl.ds(base, _ROWS_PER_W)], iidx_v)

        for p in range(2):
            def issue(g, _):
                uvec = uidx_v[pl.ds(p * half + g * 16, 16)]
                ivec = iidx_v[pl.ds(p * half + g * 16, 16)]
                for j in range(16):
                    r = uvec[j]
                    pltpu.async_copy(
                        utab_hbm.at[r >> 3, pl.ds(r & 7, 1)],
                        urows_v.at[pl.ds(g * 16 + j, 1)], usem)
                    r2 = ivec[j]
                    pltpu.async_copy(
                        itab_hbm.at[pl.ds(r2, 1)],
                        irows_v.at[pl.ds(g * 16 + j, 1)], isem)
                return 0

            lax.fori_loop(0, half // 16, issue, 0)

            def drain(i, _):
                pltpu.make_async_copy(utab_hbm.at[0, pl.ds(0, 1)],
                                      urows_v.at[pl.ds(i, 1)], usem).wait()
                pltpu.make_async_copy(itab_hbm.at[pl.ds(0, 1)],
                                      irows_v.at[pl.ds(i, 1)], isem).wait()
                return 0

            lax.fori_loop(0, half, drain, 0)
            pltpu.sync_copy(urows_v, u_out.at[pl.ds(base + p * half, half)])
            pltpu.sync_copy(irows_v, i_out.at[pl.ds(base + p * half, half)])

    return gather_kernel(user, item, utab3, itab3)


_T = 4096


def _mlp_body(u_ref, i_ref, w1u_ref, w1i_ref, b1_ref, w2_ref, b2_ref,
              w3_ref, b3_ref, o_ref):
    u16 = u_ref[...].astype(jnp.bfloat16)
    i16 = i_ref[...].astype(jnp.bfloat16)
    h = jnp.dot(u16, w1u_ref[...], preferred_element_type=jnp.float32)
    h = h + jnp.dot(i16, w1i_ref[...], preferred_element_type=jnp.float32)
    h = jnp.maximum(h + b1_ref[...], 0.0)
    h2 = jnp.dot(h.astype(jnp.bfloat16), w2_ref[...],
                 preferred_element_type=jnp.float32)
    h2 = jnp.maximum(h2 + b2_ref[...], 0.0)
    o_ref[...] = jnp.sum(h2 * w3_ref[...], axis=1) + b3_ref[0, 0]


def _tc_mlp(u, i, W1, b1, W2, b2, W3, b3):
    w1u = W1[:, :_D].T.astype(jnp.bfloat16)
    w1i = W1[:, _D:].T.astype(jnp.bfloat16)
    b1r = b1.reshape(1, 128)
    w2t = W2.T.astype(jnp.bfloat16)
    b2r = b2.reshape(1, 64)
    w3r = W3.reshape(1, 64)
    b3r = b3.reshape(1, 1)
    grid = (_B // _T,)
    full = lambda shape: pl.BlockSpec(shape, lambda b: (0, 0))
    return pl.pallas_call(
        _mlp_body,
        grid=grid,
        in_specs=[
            pl.BlockSpec((_T, _D), lambda b: (b, 0)),
            pl.BlockSpec((_T, _D), lambda b: (b, 0)),
            full((_D, 128)),
            full((_D, 128)),
            full((1, 128)),
            full((128, _D)),
            full((1, _D)),
            full((1, _D)),
            full((1, 1)),
        ],
        out_specs=pl.BlockSpec((_T,), lambda b: (b,)),
        out_shape=jax.ShapeDtypeStruct((_B,), jnp.float32),
    )(u, i, w1u, w1i, b1r, w2t, b2r, w3r, b3r)


def kernel(user, item, user_table, item_table, W1, b1, W2, b2, W3, b3):
    utab3 = user_table.reshape(125000, 8, _D)
    u, i = _sc_gather(user, item, utab3, item_table)
    return _tc_mlp(u, i, W1, b1, W2, b2, W3, b3)


# speedup vs baseline: 2.4818x; 1.0083x over previous
import functools

import jax
import jax.numpy as jnp
from jax import lax
from jax.experimental import pallas as pl
from jax.experimental.pallas import tpu as pltpu
from jax.experimental.pallas import tpu_sc as plsc

_B = 16384
_D = 64
_NC = 2
_NS = 16
_NW = _NC * _NS
_ROWS_PER_W = _B // _NW


def _sc_gather(user, item, utab3, itab3):
    mesh = plsc.VectorSubcoreMesh(core_axis_name="c", subcore_axis_name="s")
    half = _ROWS_PER_W // 2

    @functools.partial(
        pl.kernel,
        mesh=mesh,
        compiler_params=pltpu.CompilerParams(use_tc_tiling_on_sc=True),
        out_type=[
            jax.ShapeDtypeStruct((_B, _D), jnp.float32),
            jax.ShapeDtypeStruct((_B, _D), jnp.float32),
        ],
        scratch_types=[
            pltpu.VMEM((_ROWS_PER_W,), jnp.int32),
            pltpu.VMEM((_ROWS_PER_W,), jnp.int32),
            pltpu.VMEM((half, _D), jnp.float32),
            pltpu.VMEM((half, _D), jnp.float32),
            pltpu.SemaphoreType.DMA,
            pltpu.SemaphoreType.DMA,
        ],
    )
    def gather_kernel(uidx_hbm, iidx_hbm, utab_hbm, itab_hbm, u_out, i_out,
                      uidx_v, iidx_v, urows_v, irows_v, usem, isem):
        wid = lax.axis_index("s") * _NC + lax.axis_index("c")
        base = wid * _ROWS_PER_W
        pltpu.sync_copy(uidx_hbm.at[pl.ds(base, _ROWS_PER_W)], uidx_v)
        pltpu.sync_copy(iidx_hbm.at[pl.ds(base, _ROWS_PER_W)], iidx_v)

        for p in range(2):
            def issue(g, _):
                uvec = uidx_v[pl.ds(p * half + g * 16, 16)]
                ivec = iidx_v[pl.ds(p * half + g * 16, 16)]
                for j in range(16):
                    r = uvec[j]
                    pltpu.async_copy(
                        utab_hbm.at[r >> 3, pl.ds(r & 7, 1)],
                        urows_v.at[pl.ds(g * 16 + j, 1)], usem)
                    r2 = ivec[j]
                    pltpu.async_copy(
                        itab_hbm.at[pl.ds(r2, 1)],
                        irows_v.at[pl.ds(g * 16 + j, 1)], isem)
                return 0

            lax.fori_loop(0, half // 16, issue, 0)

            pltpu.make_async_copy(itab_hbm.at[pl.ds(0, half)], urows_v,
                                  usem).wait()
            pltpu.make_async_copy(itab_hbm.at[pl.ds(0, half)], irows_v,
                                  isem).wait()
            pltpu.sync_copy(urows_v, u_out.at[pl.ds(base + p * half, half)])
            pltpu.sync_copy(irows_v, i_out.at[pl.ds(base + p * half, half)])

    return gather_kernel(user, item, utab3, itab3)


_T = 4096


def _mlp_body(u_ref, i_ref, w1u_ref, w1i_ref, b1_ref, w2_ref, b2_ref,
              w3_ref, b3_ref, o_ref):
    u16 = u_ref[...].astype(jnp.bfloat16)
    i16 = i_ref[...].astype(jnp.bfloat16)
    h = jnp.dot(u16, w1u_ref[...], preferred_element_type=jnp.float32)
    h = h + jnp.dot(i16, w1i_ref[...], preferred_element_type=jnp.float32)
    h = jnp.maximum(h + b1_ref[...], 0.0)
    h2 = jnp.dot(h.astype(jnp.bfloat16), w2_ref[...],
                 preferred_element_type=jnp.float32)
    h2 = jnp.maximum(h2 + b2_ref[...], 0.0)
    o_ref[...] = jnp.sum(h2 * w3_ref[...], axis=1) + b3_ref[0, 0]


def _tc_mlp(u, i, W1, b1, W2, b2, W3, b3):
    w1u = W1[:, :_D].T.astype(jnp.bfloat16)
    w1i = W1[:, _D:].T.astype(jnp.bfloat16)
    b1r = b1.reshape(1, 128)
    w2t = W2.T.astype(jnp.bfloat16)
    b2r = b2.reshape(1, 64)
    w3r = W3.reshape(1, 64)
    b3r = b3.reshape(1, 1)
    grid = (_B // _T,)
    full = lambda shape: pl.BlockSpec(shape, lambda b: (0, 0))
    return pl.pallas_call(
        _mlp_body,
        grid=grid,
        in_specs=[
            pl.BlockSpec((_T, _D), lambda b: (b, 0)),
            pl.BlockSpec((_T, _D), lambda b: (b, 0)),
            full((_D, 128)),
            full((_D, 128)),
            full((1, 128)),
            full((128, _D)),
            full((1, _D)),
            full((1, _D)),
            full((1, 1)),
        ],
        out_specs=pl.BlockSpec((_T,), lambda b: (b,)),
        out_shape=jax.ShapeDtypeStruct((_B,), jnp.float32),
    )(u, i, w1u, w1i, b1r, w2t, b2r, w3r, b3r)


def kernel(user, item, user_table, item_table, W1, b1, W2, b2, W3, b3):
    utab3 = user_table.reshape(125000, 8, _D)
    u, i = _sc_gather(user, item, utab3, item_table)
    return _tc_mlp(u, i, W1, b1, W2, b2, W3, b3)


# speedup vs baseline: 2.4897x; 1.0032x over previous
import functools

import jax
import jax.numpy as jnp
from jax import lax
from jax.experimental import pallas as pl
from jax.experimental.pallas import tpu as pltpu
from jax.experimental.pallas import tpu_sc as plsc

_B = 16384
_D = 64
_NC = 2
_NS = 16
_NW = _NC * _NS
_ROWS_PER_W = _B // _NW


def _sc_gather(user, item, utab3, itab3):
    mesh = plsc.VectorSubcoreMesh(core_axis_name="c", subcore_axis_name="s")
    half = _ROWS_PER_W // 2

    @functools.partial(
        pl.kernel,
        mesh=mesh,
        compiler_params=pltpu.CompilerParams(use_tc_tiling_on_sc=True),
        out_type=[
            jax.ShapeDtypeStruct((_B, _D), jnp.float32),
            jax.ShapeDtypeStruct((_B, _D), jnp.float32),
        ],
        scratch_types=[
            pltpu.VMEM((_ROWS_PER_W,), jnp.int32),
            pltpu.VMEM((_ROWS_PER_W,), jnp.int32),
            pltpu.VMEM((half, _D), jnp.float32),
            pltpu.VMEM((half, _D), jnp.float32),
            pltpu.SemaphoreType.DMA,
            pltpu.SemaphoreType.DMA,
        ],
    )
    def gather_kernel(uidx_hbm, iidx_hbm, utab_hbm, itab_hbm, u_out, i_out,
                      uidx_v, iidx_v, urows_v, irows_v, usem, isem):
        wid = lax.axis_index("s") * _NC + lax.axis_index("c")
        base = wid * _ROWS_PER_W
        pltpu.sync_copy(uidx_hbm.at[pl.ds(base, _ROWS_PER_W)], uidx_v)
        pltpu.sync_copy(iidx_hbm.at[pl.ds(base, _ROWS_PER_W)], iidx_v)

        for p in range(2):
            def issue(g, _):
                uvec = uidx_v[pl.ds(p * half + g * 16, 16)]
                ivec = iidx_v[pl.ds(p * half + g * 16, 16)]
                for j in range(16):
                    r = uvec[j]
                    pltpu.async_copy(
                        utab_hbm.at[r >> 3, pl.ds(r & 7, 1)],
                        urows_v.at[pl.ds(g * 16 + j, 1)], usem)
                    r2 = ivec[j]
                    pltpu.async_copy(
                        itab_hbm.at[pl.ds(r2, 1)],
                        irows_v.at[pl.ds(g * 16 + j, 1)], isem)
                return 0

            lax.fori_loop(0, half // 16, issue, 0)

            pltpu.make_async_copy(itab_hbm.at[pl.ds(0, half)], urows_v,
                                  usem).wait()
            pltpu.make_async_copy(itab_hbm.at[pl.ds(0, half)], irows_v,
                                  isem).wait()
            pltpu.sync_copy(urows_v, u_out.at[pl.ds(base + p * half, half)])
            pltpu.sync_copy(irows_v, i_out.at[pl.ds(base + p * half, half)])

    return gather_kernel(user, item, utab3, itab3)


_T = 8192


def _mlp_body(u_ref, i_ref, w1u_ref, w1i_ref, b1_ref, w2_ref, b2_ref,
              w3_ref, b3_ref, o_ref):
    u16 = u_ref[...].astype(jnp.bfloat16)
    i16 = i_ref[...].astype(jnp.bfloat16)
    h = jnp.dot(u16, w1u_ref[...], preferred_element_type=jnp.float32)
    h = h + jnp.dot(i16, w1i_ref[...], preferred_element_type=jnp.float32)
    h = jnp.maximum(h + b1_ref[...], 0.0)
    h2 = jnp.dot(h.astype(jnp.bfloat16), w2_ref[...],
                 preferred_element_type=jnp.float32)
    h2 = jnp.maximum(h2 + b2_ref[...], 0.0)
    o_ref[...] = jnp.sum(h2 * w3_ref[...], axis=1) + b3_ref[0, 0]


def _tc_mlp(u, i, W1, b1, W2, b2, W3, b3):
    w1u = W1[:, :_D].T.astype(jnp.bfloat16)
    w1i = W1[:, _D:].T.astype(jnp.bfloat16)
    b1r = b1.reshape(1, 128)
    w2t = W2.T.astype(jnp.bfloat16)
    b2r = b2.reshape(1, 64)
    w3r = W3.reshape(1, 64)
    b3r = b3.reshape(1, 1)
    grid = (_B // _T,)
    full = lambda shape: pl.BlockSpec(shape, lambda b: (0, 0))
    return pl.pallas_call(
        _mlp_body,
        grid=grid,
        in_specs=[
            pl.BlockSpec((_T, _D), lambda b: (b, 0)),
            pl.BlockSpec((_T, _D), lambda b: (b, 0)),
            full((_D, 128)),
            full((_D, 128)),
            full((1, 128)),
            full((128, _D)),
            full((1, _D)),
            full((1, _D)),
            full((1, 1)),
        ],
        out_specs=pl.BlockSpec((_T,), lambda b: (b,)),
        out_shape=jax.ShapeDtypeStruct((_B,), jnp.float32),
    )(u, i, w1u, w1i, b1r, w2t, b2r, w3r, b3r)


def kernel(user, item, user_table, item_table, W1, b1, W2, b2, W3, b3):
    utab3 = user_table.reshape(125000, 8, _D)
    u, i = _sc_gather(user, item, utab3, item_table)
    return _tc_mlp(u, i, W1, b1, W2, b2, W3, b3)
